# baseline (speedup 1.0000x reference)
import jax
import jax.numpy as jnp
from jax import lax
from jax.experimental import pallas as pl
from jax.experimental.pallas import tpu as pltpu
from jax.experimental.pallas import tpu_sc as plsc

NC = 1
NS = 16
L = 16
NW = NC * NS

B = 128
D = 128
CH = D // L
BPW = B // NW


def _rsqrt_nr(x):
    i = lax.bitcast_convert_type(x, jnp.int32)
    i = jnp.int32(0x5F3759DF) - (i >> 1)
    y = lax.bitcast_convert_type(i, jnp.float32)
    for _ in range(3):
        y = y * (jnp.float32(1.5) - jnp.float32(0.5) * x * y * y)
    return y


def _body(trip_hbm, ent_hbm, rel_hbm,
          out_hbm,
          trip_v, tidx_v, hidx_v, ridx_v,
          t_rows, h_rows, r_rows,
          pub_v, tn_local, shared_tn,
          posd_s, negd_s, loss_s,
          sem_t, sem_h, sem_r):
    sid = lax.axis_index("s")
    base = sid * BPW

    pltpu.sync_copy(trip_hbm, trip_v)

    iot = lax.iota(jnp.int32, 16)
    lane7 = jnp.bitwise_and(iot, 7)
    is_lo8 = iot < 8

    flat = (base + lane7) * 3 + jnp.where(is_lo8, 0, 3 * B)
    tidx_v[...] = plsc.load_gather(trip_v, [flat + 2])
    cp_t = pltpu.async_copy(ent_hbm.at[tidx_v], t_rows, sem_t)

    hidx_v[...] = plsc.load_gather(trip_v, [flat])
    ridx_v[...] = plsc.load_gather(trip_v, [flat + 1])

    cp_h = pltpu.async_copy(ent_hbm.at[hidx_v], h_rows, sem_h)
    cp_r = pltpu.async_copy(rel_hbm.at[ridx_v], r_rows, sem_r)

    cp_t.wait()

    def p1_body(i, pub):
        acc2 = jnp.zeros((16,), jnp.float32)
        acc1 = jnp.zeros((16,), jnp.float32)
        for c in range(CH):
            v = t_rows[i, pl.ds(c * L, L)]
            acc2 = acc2 + v * v
            acc1 = acc1 + jnp.abs(v)
        tn_i = jnp.sum(acc1) * _rsqrt_nr(
            jnp.maximum(jnp.sum(acc2), jnp.float32(1e-24)))
        return jnp.where(iot == i, tn_i, pub)

    pub_v[...] = lax.fori_loop(0, 16, p1_body, jnp.zeros((16,), jnp.float32))
    pltpu.sync_copy(pub_v, shared_tn.at[sid])

    cp_h.wait()
    cp_r.wait()

    def hn_body(i, carry):
        acc2 = jnp.zeros((16,), jnp.float32)
        for c in range(CH):
            v = h_rows[i, pl.ds(c * L, L)]
            acc2 = acc2 + v * v
        s = _rsqrt_nr(jnp.maximum(jnp.sum(acc2), jnp.float32(1e-24)))
        for c in range(CH):
            sl = pl.ds(c * L, L)
            h_rows[i, sl] = h_rows[i, sl] * s
        return carry

    lax.fori_loop(0, 16, hn_body, jnp.int32(0))

    plsc.subcore_barrier()
    pltpu.sync_copy(shared_tn, tn_local)

    tnp = []
    tnn = []
    for c in range(CH):
        j = iot + c * L
        w = lax.shift_right_logical(j, 3)
        l8 = jnp.bitwise_and(j, 7)
        tnp.append(plsc.load_gather(tn_local, [w, l8]))
        tnn.append(plsc.load_gather(tn_local, [w, l8 + 8]))

    def out_body(i, carry):
        for c in range(CH):
            sl = pl.ds(c * L, L)
            pd = h_rows[i, sl] + r_rows[i, sl] - tnp[c]
            nd = h_rows[8 + i, sl] + r_rows[8 + i, sl] - tnn[c]
            posd_s[i, sl] = pd
            negd_s[i, sl] = nd
            loss_s[i, sl] = jnp.maximum(pd - nd + jnp.float32(1.0),
                                        jnp.float32(0.0))
        return carry

    lax.fori_loop(0, BPW, out_body, jnp.int32(0))

    cp_o0 = pltpu.async_copy(loss_s, out_hbm.at[0, pl.ds(base, BPW)], sem_t)
    cp_o1 = pltpu.async_copy(posd_s, out_hbm.at[1, pl.ds(base, BPW)], sem_h)
    cp_o2 = pltpu.async_copy(negd_s, out_hbm.at[2, pl.ds(base, BPW)], sem_r)
    cp_o0.wait()
    cp_o1.wait()
    cp_o2.wait()


def kernel(positive_triplets, negative_triplets, entities_emb, relations_emb):
    out = jax.ShapeDtypeStruct((3, B, D), jnp.float32)
    f = pl.kernel(
        _body,
        out_type=out,
        mesh=plsc.VectorSubcoreMesh(core_axis_name="c", subcore_axis_name="s",
                                    num_cores=NC, num_subcores=NS),
        compiler_params=pltpu.CompilerParams(needs_layout_passes=False,
                                             skip_device_barrier=True,
                                             disable_bounds_checks=True,
                                             disable_semaphore_checks=True),
        scratch_types=[
            pltpu.VMEM((2 * B * 3,), jnp.int32),
            pltpu.VMEM((16,), jnp.int32),
            pltpu.VMEM((16,), jnp.int32),
            pltpu.VMEM((16,), jnp.int32),
            pltpu.VMEM((16, D), jnp.float32),
            pltpu.VMEM((16, D), jnp.float32),
            pltpu.VMEM((16, D), jnp.float32),
            pltpu.VMEM((16,), jnp.float32),
            pltpu.VMEM((NS, 16), jnp.float32),
            pltpu.VMEM_SHARED((NS, 16), jnp.float32),
            pltpu.VMEM((BPW, D), jnp.float32),
            pltpu.VMEM((BPW, D), jnp.float32),
            pltpu.VMEM((BPW, D), jnp.float32),
            pltpu.SemaphoreType.DMA,
            pltpu.SemaphoreType.DMA,
            pltpu.SemaphoreType.DMA,
        ],
    )
    trips = jnp.concatenate([positive_triplets.reshape(-1),
                             negative_triplets.reshape(-1)])
    packed = f(trips, entities_emb, relations_emb)
    return (packed[0], packed[1], packed[2])


# speedup vs baseline: 1.0648x; 1.0648x over previous
import jax
import jax.numpy as jnp
from jax import lax
from jax.experimental import pallas as pl
from jax.experimental.pallas import tpu as pltpu
from jax.experimental.pallas import tpu_sc as plsc

NC = 1
NS = 16
L = 16
NW = NC * NS

B = 128
D = 128
CH = D // L
BPW = B // NW


def _rsqrt_nr(x):
    i = lax.bitcast_convert_type(x, jnp.int32)
    i = jnp.int32(0x5F3759DF) - (i >> 1)
    y = lax.bitcast_convert_type(i, jnp.float32)
    for _ in range(3):
        y = y * (jnp.float32(1.5) - jnp.float32(0.5) * x * y * y)
    return y


def _body(trip_hbm, ent_hbm, rel_hbm,
          loss_hbm, posd_hbm, negd_hbm,
          trip_v, tidx_v, hidx_v, ridx_v,
          t_rows, h_rows, r_rows,
          pub_v, tn_local, shared_tn,
          posd_s, negd_s, loss_s,
          sem_t, sem_h, sem_r):
    sid = lax.axis_index("s")
    base = sid * BPW

    pltpu.sync_copy(trip_hbm, trip_v)

    iot = lax.iota(jnp.int32, 16)
    lane7 = jnp.bitwise_and(iot, 7)
    is_lo8 = iot < 8

    flat = (base + lane7) * 3 + jnp.where(is_lo8, 0, 3 * B)
    tidx_v[...] = plsc.load_gather(trip_v, [flat + 2])
    cp_t = pltpu.async_copy(ent_hbm.at[tidx_v], t_rows, sem_t)

    hidx_v[...] = plsc.load_gather(trip_v, [flat])
    ridx_v[...] = plsc.load_gather(trip_v, [flat + 1])

    cp_h = pltpu.async_copy(ent_hbm.at[hidx_v], h_rows, sem_h)
    cp_r = pltpu.async_copy(rel_hbm.at[ridx_v], r_rows, sem_r)

    cp_t.wait()
    pub = jnp.zeros((16,), jnp.float32)
    for i in range(16):
        acc2 = jnp.zeros((16,), jnp.float32)
        acc1 = jnp.zeros((16,), jnp.float32)
        for c in range(CH):
            v = t_rows[i, pl.ds(c * L, L)]
            acc2 = acc2 + v * v
            acc1 = acc1 + jnp.abs(v)
        tn_i = jnp.sum(acc1) * _rsqrt_nr(
            jnp.maximum(jnp.sum(acc2), jnp.float32(1e-24)))
        pub = jnp.where(iot == i, tn_i, pub)
    pub_v[...] = pub
    pltpu.sync_copy(pub_v, shared_tn.at[sid])

    cp_h.wait()
    cp_r.wait()
    hscale = []
    for i in range(16):
        acc2 = jnp.zeros((16,), jnp.float32)
        for c in range(CH):
            v = h_rows[i, pl.ds(c * L, L)]
            acc2 = acc2 + v * v
        hscale.append(_rsqrt_nr(jnp.maximum(jnp.sum(acc2), jnp.float32(1e-24))))

    plsc.subcore_barrier()
    pltpu.sync_copy(shared_tn, tn_local)

    for c in range(CH):
        j = iot + c * L
        w = lax.shift_right_logical(j, 3)
        l8 = jnp.bitwise_and(j, 7)
        tnp_c = plsc.load_gather(tn_local, [w, l8])
        tnn_c = plsc.load_gather(tn_local, [w, l8 + 8])
        for i in range(BPW):
            sl = pl.ds(c * L, L)
            pd = h_rows[i, sl] * hscale[i] + r_rows[i, sl] - tnp_c
            nd = h_rows[8 + i, sl] * hscale[8 + i] + r_rows[8 + i, sl] - tnn_c
            posd_s[i, sl] = pd
            negd_s[i, sl] = nd
            loss_s[i, sl] = jnp.maximum(pd - nd + jnp.float32(1.0),
                                        jnp.float32(0.0))

    cp_o0 = pltpu.async_copy(loss_s, loss_hbm.at[pl.ds(base, BPW)], sem_t)
    cp_o1 = pltpu.async_copy(posd_s, posd_hbm.at[pl.ds(base, BPW)], sem_h)
    cp_o2 = pltpu.async_copy(negd_s, negd_hbm.at[pl.ds(base, BPW)], sem_r)
    cp_o0.wait()
    cp_o1.wait()
    cp_o2.wait()


def kernel(positive_triplets, negative_triplets, entities_emb, relations_emb):
    out = jax.ShapeDtypeStruct((B, D), jnp.float32)
    f = pl.kernel(
        _body,
        out_type=(out, out, out),
        mesh=plsc.VectorSubcoreMesh(core_axis_name="c", subcore_axis_name="s",
                                    num_cores=NC, num_subcores=NS),
        compiler_params=pltpu.CompilerParams(needs_layout_passes=False,
                                             skip_device_barrier=True,
                                             disable_bounds_checks=True,
                                             disable_semaphore_checks=True),
        scratch_types=[
            pltpu.VMEM((2 * B * 3,), jnp.int32),
            pltpu.VMEM((16,), jnp.int32),
            pltpu.VMEM((16,), jnp.int32),
            pltpu.VMEM((16,), jnp.int32),
            pltpu.VMEM((16, D), jnp.float32),
            pltpu.VMEM((16, D), jnp.float32),
            pltpu.VMEM((16, D), jnp.float32),
            pltpu.VMEM((16,), jnp.float32),
            pltpu.VMEM((NS, 16), jnp.float32),
            pltpu.VMEM_SHARED((NS, 16), jnp.float32),
            pltpu.VMEM((BPW, D), jnp.float32),
            pltpu.VMEM((BPW, D), jnp.float32),
            pltpu.VMEM((BPW, D), jnp.float32),
            pltpu.SemaphoreType.DMA,
            pltpu.SemaphoreType.DMA,
            pltpu.SemaphoreType.DMA,
        ],
    )
    trips = jnp.concatenate([positive_triplets.reshape(-1),
                             negative_triplets.reshape(-1)])
    return f(trips, entities_emb, relations_emb)
